# baseline (speedup 1.0000x reference)
import functools

import jax
import jax.numpy as jnp
from jax import lax
from jax.experimental import pallas as pl
from jax.experimental.pallas import tpu as pltpu
from jax.experimental.pallas import tpu_sc as plsc

_NUM_EXPERTS = 16
_B = 2
_S = 4096
_T = _B * _S
_NC = 2
_NS = 16
_NW = _NC * _NS
_WPB = _NW // _B

_SPB = 256
_TPW = _SPB // _WPB
_TCB = _S - _SPB
_UNROLL = 4

_K1 = 0x00000000
_K2 = 0x0000002A
_K3 = _K1 ^ _K2 ^ 0x1BD11BDA
_KS = (_K1, _K2, _K3)
_ROT = ((13, 15, 26, 6), (17, 29, 16, 24))

_TINY = 1.1754943508222875e-38
_LN2 = 0.6931471805599453

_NR_HI = -6.153845310211182
_NR_LO = -15.999996185302734

_LOG_COEFS = (
    0.9999951124191284,
    -0.49984779953956604,
    0.33161383867263794,
    -0.24010024964809418,
    0.16648142039775848,
    -0.09413309395313263,
    0.0354582816362381,
    -0.006320376414805651,
)


def _rotl(x, d):
    return (x << jnp.uint32(d)) | (x >> jnp.uint32(32 - d))


def _threefry_bits(c1):
    x0 = jnp.zeros_like(c1) + jnp.uint32(_KS[0])
    x1 = c1 + jnp.uint32(_KS[1])
    for i in range(5):
        for d in _ROT[i % 2]:
            x0 = x0 + x1
            x1 = _rotl(x1, d)
            x1 = x0 ^ x1
        x0 = x0 + jnp.uint32(_KS[(i + 1) % 3])
        x1 = x1 + jnp.uint32(_KS[(i + 2) % 3] + i + 1)
    return x0 ^ x1


def _vlog(v):
    bv = lax.bitcast_convert_type(v, jnp.uint32)
    e_i = lax.convert_element_type(bv >> jnp.uint32(23), jnp.int32) - jnp.int32(127)
    e = lax.convert_element_type(e_i, jnp.float32)
    m = lax.bitcast_convert_type(
        (bv & jnp.uint32(0x007FFFFF)) | jnp.uint32(0x3F800000), jnp.float32)
    x = m - jnp.float32(1.0)
    p = jnp.zeros_like(x)
    for c in _LOG_COEFS[::-1]:
        p = p * x + jnp.float32(c)
    p = p * x
    return e * jnp.float32(_LN2) + p


def _argmin_key(c1, lane, rneg):
    bits = _threefry_bits(c1)
    fb = (bits >> jnp.uint32(9)) | jnp.uint32(0x3F800000)
    f = lax.bitcast_convert_type(fb, jnp.float32) - jnp.float32(1.0)
    u = f + jnp.float32(_TINY)
    val = _vlog(u) * rneg
    return (lax.bitcast_convert_type(val, jnp.int32) & jnp.int32(-16)) | lane


def _lane_min(x, perms):
    for p in perms:
        x = jnp.minimum(x, x.at[p].get(mode="promise_in_bounds"))
    return x


@functools.partial(
    pl.kernel,
    out_type=(
        jax.ShapeDtypeStruct((_B * _NUM_EXPERTS * _SPB,), jnp.float32),
        jax.ShapeDtypeStruct((_T,), jnp.float32),
    ),
    mesh=plsc.VectorSubcoreMesh(core_axis_name="c", subcore_axis_name="s"),
    scratch_types=[
        pltpu.VMEM((_NUM_EXPERTS * _TPW,), jnp.float32),
        pltpu.VMEM((_T // _NW,), jnp.float32),
        pltpu.SemaphoreType.DMA,
    ],
)
def _router_sc(out_oh, out_ones, oh_v, ones_v, sem):
    wid = lax.axis_index("s") * _NC + lax.axis_index("c")
    lane_i32 = lax.iota(jnp.int32, 16)
    lane_u32 = lax.convert_element_type(lane_i32, jnp.uint32)
    rneg = jnp.where(lane_i32 < jnp.int32(2),
                     jnp.float32(_NR_HI), jnp.float32(_NR_LO))
    perms = tuple(lane_i32 ^ jnp.int32(1 << k) for k in range(4))
    ones = jnp.full((16,), 1.0, dtype=jnp.float32)
    for j in range(_T // _NW // 16):
        ones_v[pl.ds(j * 16, 16)] = ones
    b = wid // _WPB
    w16 = wid % _WPB
    t0 = b * _S + w16 * _TPW

    def step16(s2, carry):
        i0 = s2 * 16

        def step4(s4, acc):
            jb = s4 * _UNROLL
            for k in range(_UNROLL):
                j = jb + k
                t_u32 = lax.convert_element_type(t0 + (i0 + j), jnp.uint32)
                c1 = lane_u32 + jnp.uint32(16) * t_u32
                key = _argmin_key(c1, lane_i32, rneg)
                idx = _lane_min(key, perms) & jnp.int32(15)
                acc = jnp.where(lane_i32 == j, idx, acc)
            return acc

        idxvec = lax.fori_loop(0, 16 // _UNROLL, step4,
                               jnp.zeros((16,), jnp.int32))
        for e in range(_NUM_EXPERTS):
            row = jnp.where(idxvec == jnp.int32(e),
                            jnp.float32(1.0), jnp.float32(0.0))
            oh_v[pl.ds(e * _TPW + i0, 16)] = row
        return carry

    lax.fori_loop(0, _TPW // 16, step16, jnp.int32(0))
    flat_base = b * (_NUM_EXPERTS * _SPB) + w16 * _TPW
    copies = []
    for e in range(_NUM_EXPERTS):
        copies.append(pltpu.async_copy(
            oh_v.at[pl.ds(e * _TPW, _TPW)],
            out_oh.at[pl.ds(flat_base + e * _SPB, _TPW)], sem))
    copies.append(pltpu.async_copy(
        ones_v, out_ones.at[pl.ds(wid * (_T // _NW), _T // _NW)], sem))
    for c in copies:
        c.wait()


def _tc_body(out_ref):
    for b in range(_B):
        col = lax.broadcasted_iota(jnp.int32, (_NUM_EXPERTS, _TCB), 1)
        row = lax.broadcasted_iota(jnp.int32, (_NUM_EXPERTS, _TCB), 0)
        t = jnp.int32(b * _S + _SPB) + col
        c1 = lax.convert_element_type(jnp.int32(16) * t + row, jnp.uint32)
        rneg = jnp.where(row < jnp.int32(2),
                         jnp.float32(_NR_HI), jnp.float32(_NR_LO))
        key = _argmin_key(c1, row, rneg)
        idx = jnp.min(key, axis=0, keepdims=True) & jnp.int32(15)
        out_ref[b, :, pl.ds(_SPB, _TCB)] = jnp.where(
            row == idx, jnp.float32(1.0), jnp.float32(0.0))


_tc_router = pl.pallas_call(
    _tc_body,
    out_shape=jax.ShapeDtypeStruct((_B, _NUM_EXPERTS, _S), jnp.float32),
)


def kernel(x):
    sc_oh, sc_ones = _router_sc()
    tc_oh = _tc_router()
    strip = sc_oh.reshape(_B, _NUM_EXPERTS, _SPB)
    full = lax.dynamic_update_slice(tc_oh, strip, (0, 0, 0))
    one_hot = jnp.transpose(full, (0, 2, 1))
    router_probabilities = sc_ones.reshape(_B, _S, 1).astype(x.dtype)
    return (one_hot, router_probabilities, one_hot)


# speedup vs baseline: 1.0551x; 1.0551x over previous
import functools

import jax
import jax.numpy as jnp
from jax import lax
from jax.experimental import pallas as pl
from jax.experimental.pallas import tpu as pltpu
from jax.experimental.pallas import tpu_sc as plsc

_NUM_EXPERTS = 16
_B = 2
_S = 4096
_T = _B * _S
_NC = 1
_NS = 16
_NW = _NC * _NS
_WPB = _NW // _B

_SPB = 256
_TPW = _SPB // _WPB
_TCB = _S - _SPB
_UNROLL = 4

_K1 = 0x00000000
_K2 = 0x0000002A
_K3 = _K1 ^ _K2 ^ 0x1BD11BDA
_KS = (_K1, _K2, _K3)
_ROT = ((13, 15, 26, 6), (17, 29, 16, 24))

_TINY = 1.1754943508222875e-38
_LN2 = 0.6931471805599453

_NR_HI = -6.153845310211182
_NR_LO = -15.999996185302734

_LOG_COEFS = (
    0.9999951124191284,
    -0.49984779953956604,
    0.33161383867263794,
    -0.24010024964809418,
    0.16648142039775848,
    -0.09413309395313263,
    0.0354582816362381,
    -0.006320376414805651,
)


def _rotl(x, d):
    return (x << jnp.uint32(d)) | (x >> jnp.uint32(32 - d))


def _threefry_bits(c1):
    x0 = jnp.zeros_like(c1) + jnp.uint32(_KS[0])
    x1 = c1 + jnp.uint32(_KS[1])
    for i in range(5):
        for d in _ROT[i % 2]:
            x0 = x0 + x1
            x1 = _rotl(x1, d)
            x1 = x0 ^ x1
        x0 = x0 + jnp.uint32(_KS[(i + 1) % 3])
        x1 = x1 + jnp.uint32(_KS[(i + 2) % 3] + i + 1)
    return x0 ^ x1


def _vlog(v):
    bv = lax.bitcast_convert_type(v, jnp.uint32)
    e_i = lax.convert_element_type(bv >> jnp.uint32(23), jnp.int32) - jnp.int32(127)
    e = lax.convert_element_type(e_i, jnp.float32)
    m = lax.bitcast_convert_type(
        (bv & jnp.uint32(0x007FFFFF)) | jnp.uint32(0x3F800000), jnp.float32)
    x = m - jnp.float32(1.0)
    p = jnp.zeros_like(x)
    for c in _LOG_COEFS[::-1]:
        p = p * x + jnp.float32(c)
    p = p * x
    return e * jnp.float32(_LN2) + p


def _argmin_key(c1, lane, rneg):
    bits = _threefry_bits(c1)
    fb = (bits >> jnp.uint32(9)) | jnp.uint32(0x3F800000)
    f = lax.bitcast_convert_type(fb, jnp.float32) - jnp.float32(1.0)
    u = f + jnp.float32(_TINY)
    val = _vlog(u) * rneg
    return (lax.bitcast_convert_type(val, jnp.int32) & jnp.int32(-16)) | lane


def _lane_min(x, perms):
    for p in perms:
        x = jnp.minimum(x, x.at[p].get(mode="promise_in_bounds"))
    return x


@functools.partial(
    pl.kernel,
    out_type=(
        jax.ShapeDtypeStruct((_B * _NUM_EXPERTS * _SPB,), jnp.float32),
        jax.ShapeDtypeStruct((_T,), jnp.float32),
    ),
    mesh=plsc.VectorSubcoreMesh(core_axis_name="c", subcore_axis_name="s",
                                num_cores=_NC),
    scratch_types=[
        pltpu.VMEM((_NUM_EXPERTS * _TPW,), jnp.float32),
        pltpu.VMEM((_T // _NW,), jnp.float32),
        pltpu.SemaphoreType.DMA,
    ],
)
def _router_sc(out_oh, out_ones, oh_v, ones_v, sem):
    wid = lax.axis_index("s") * _NC + lax.axis_index("c")
    lane_i32 = lax.iota(jnp.int32, 16)
    lane_u32 = lax.convert_element_type(lane_i32, jnp.uint32)
    rneg = jnp.where(lane_i32 < jnp.int32(2),
                     jnp.float32(_NR_HI), jnp.float32(_NR_LO))
    perms = tuple(lane_i32 ^ jnp.int32(1 << k) for k in range(4))
    ones = jnp.full((16,), 1.0, dtype=jnp.float32)
    for j in range(_T // _NW // 16):
        ones_v[pl.ds(j * 16, 16)] = ones
    b = wid // _WPB
    w16 = wid % _WPB
    t0 = b * _S + w16 * _TPW

    def step16(s2, carry):
        i0 = s2 * 16

        def step4(s4, acc):
            jb = s4 * _UNROLL
            for k in range(_UNROLL):
                j = jb + k
                t_u32 = lax.convert_element_type(t0 + (i0 + j), jnp.uint32)
                c1 = lane_u32 + jnp.uint32(16) * t_u32
                key = _argmin_key(c1, lane_i32, rneg)
                idx = _lane_min(key, perms) & jnp.int32(15)
                acc = jnp.where(lane_i32 == j, idx, acc)
            return acc

        idxvec = lax.fori_loop(0, 16 // _UNROLL, step4,
                               jnp.zeros((16,), jnp.int32))
        for e in range(_NUM_EXPERTS):
            row = jnp.where(idxvec == jnp.int32(e),
                            jnp.float32(1.0), jnp.float32(0.0))
            oh_v[pl.ds(e * _TPW + i0, 16)] = row
        return carry

    lax.fori_loop(0, _TPW // 16, step16, jnp.int32(0))
    flat_base = b * (_NUM_EXPERTS * _SPB) + w16 * _TPW
    copies = []
    for e in range(_NUM_EXPERTS):
        copies.append(pltpu.async_copy(
            oh_v.at[pl.ds(e * _TPW, _TPW)],
            out_oh.at[pl.ds(flat_base + e * _SPB, _TPW)], sem))
    copies.append(pltpu.async_copy(
        ones_v, out_ones.at[pl.ds(wid * (_T // _NW), _T // _NW)], sem))
    for c in copies:
        c.wait()


def _tc_body(out_ref):
    for b in range(_B):
        col = lax.broadcasted_iota(jnp.int32, (_NUM_EXPERTS, _TCB), 1)
        row = lax.broadcasted_iota(jnp.int32, (_NUM_EXPERTS, _TCB), 0)
        t = jnp.int32(b * _S + _SPB) + col
        c1 = lax.convert_element_type(jnp.int32(16) * t + row, jnp.uint32)
        rneg = jnp.where(row < jnp.int32(2),
                         jnp.float32(_NR_HI), jnp.float32(_NR_LO))
        key = _argmin_key(c1, row, rneg)
        idx = jnp.min(key, axis=0, keepdims=True) & jnp.int32(15)
        out_ref[b] = jnp.where(row == idx, jnp.float32(1.0), jnp.float32(0.0))


_tc_router = pl.pallas_call(
    _tc_body,
    out_shape=jax.ShapeDtypeStruct((_B, _NUM_EXPERTS, _TCB), jnp.float32),
)


def kernel(x):
    sc_oh, sc_ones = _router_sc()
    tc_oh = _tc_router()
    full = jnp.concatenate(
        [sc_oh.reshape(_B, _NUM_EXPERTS, _SPB), tc_oh], axis=2)
    one_hot = jnp.transpose(full, (0, 2, 1))
    router_probabilities = sc_ones.reshape(_B, _S, 1).astype(x.dtype)
    return (one_hot, router_probabilities, one_hot)


# speedup vs baseline: 1.1217x; 1.0632x over previous
import functools

import jax
import jax.numpy as jnp
from jax import lax
from jax.experimental import pallas as pl
from jax.experimental.pallas import tpu as pltpu
from jax.experimental.pallas import tpu_sc as plsc

_NUM_EXPERTS = 16
_B = 2
_S = 4096
_T = _B * _S
_NC = 1
_NS = 16
_NW = _NC * _NS
_WPB = _NW // _B

_SPB = 128
_TPW = _SPB // _WPB
_TCB = _S - _SPB
_UNROLL = 4

_K1 = 0x00000000
_K2 = 0x0000002A
_K3 = _K1 ^ _K2 ^ 0x1BD11BDA
_KS = (_K1, _K2, _K3)
_ROT = ((13, 15, 26, 6), (17, 29, 16, 24))

_TINY = 1.1754943508222875e-38
_LN2 = 0.6931471805599453

_NR_HI = -6.153845310211182
_NR_LO = -15.999996185302734

_LOG_COEFS = (
    0.9999951124191284,
    -0.49984779953956604,
    0.33161383867263794,
    -0.24010024964809418,
    0.16648142039775848,
    -0.09413309395313263,
    0.0354582816362381,
    -0.006320376414805651,
)


def _rotl(x, d):
    return (x << jnp.uint32(d)) | (x >> jnp.uint32(32 - d))


def _threefry_bits(c1):
    x0 = jnp.zeros_like(c1) + jnp.uint32(_KS[0])
    x1 = c1 + jnp.uint32(_KS[1])
    for i in range(5):
        for d in _ROT[i % 2]:
            x0 = x0 + x1
            x1 = _rotl(x1, d)
            x1 = x0 ^ x1
        x0 = x0 + jnp.uint32(_KS[(i + 1) % 3])
        x1 = x1 + jnp.uint32(_KS[(i + 2) % 3] + i + 1)
    return x0 ^ x1


def _vlog(v):
    bv = lax.bitcast_convert_type(v, jnp.uint32)
    e_i = lax.convert_element_type(bv >> jnp.uint32(23), jnp.int32) - jnp.int32(127)
    e = lax.convert_element_type(e_i, jnp.float32)
    m = lax.bitcast_convert_type(
        (bv & jnp.uint32(0x007FFFFF)) | jnp.uint32(0x3F800000), jnp.float32)
    x = m - jnp.float32(1.0)
    p = jnp.zeros_like(x)
    for c in _LOG_COEFS[::-1]:
        p = p * x + jnp.float32(c)
    p = p * x
    return e * jnp.float32(_LN2) + p


def _argmin_key(c1, lane, rneg):
    bits = _threefry_bits(c1)
    fb = (bits >> jnp.uint32(9)) | jnp.uint32(0x3F800000)
    f = lax.bitcast_convert_type(fb, jnp.float32) - jnp.float32(1.0)
    u = f + jnp.float32(_TINY)
    val = _vlog(u) * rneg
    return (lax.bitcast_convert_type(val, jnp.int32) & jnp.int32(-16)) | lane


def _lane_min(x, perms):
    for p in perms:
        x = jnp.minimum(x, x.at[p].get(mode="promise_in_bounds"))
    return x


@functools.partial(
    pl.kernel,
    out_type=(
        jax.ShapeDtypeStruct((_B * _NUM_EXPERTS * _SPB,), jnp.float32),
        jax.ShapeDtypeStruct((_T,), jnp.float32),
    ),
    mesh=plsc.VectorSubcoreMesh(core_axis_name="c", subcore_axis_name="s",
                                num_cores=_NC),
    scratch_types=[
        pltpu.VMEM((_NUM_EXPERTS * _TPW,), jnp.float32),
        pltpu.VMEM((_T // _NW,), jnp.float32),
        pltpu.SemaphoreType.DMA,
    ],
)
def _router_sc(out_oh, out_ones, oh_v, ones_v, sem):
    wid = lax.axis_index("s") * _NC + lax.axis_index("c")
    lane_i32 = lax.iota(jnp.int32, 16)
    lane_u32 = lax.convert_element_type(lane_i32, jnp.uint32)
    rneg = jnp.where(lane_i32 < jnp.int32(2),
                     jnp.float32(_NR_HI), jnp.float32(_NR_LO))
    perms = tuple(lane_i32 ^ jnp.int32(1 << k) for k in range(4))
    ones = jnp.full((16,), 1.0, dtype=jnp.float32)
    for j in range(_T // _NW // 16):
        ones_v[pl.ds(j * 16, 16)] = ones
    b = wid // _WPB
    w16 = wid % _WPB
    t0 = b * _S + w16 * _TPW

    def step16(s2, carry):
        i0 = s2 * 16

        def step4(s4, acc):
            jb = s4 * _UNROLL
            for k in range(_UNROLL):
                j = jb + k
                t_u32 = lax.convert_element_type(t0 + (i0 + j), jnp.uint32)
                c1 = lane_u32 + jnp.uint32(16) * t_u32
                key = _argmin_key(c1, lane_i32, rneg)
                idx = _lane_min(key, perms) & jnp.int32(15)
                acc = jnp.where(lane_i32 == j, idx, acc)
            return acc

        idxvec = lax.fori_loop(0, 16 // _UNROLL, step4,
                               jnp.zeros((16,), jnp.int32))
        for e in range(_NUM_EXPERTS):
            row = jnp.where(idxvec == jnp.int32(e),
                            jnp.float32(1.0), jnp.float32(0.0))
            oh_v[pl.ds(e * _TPW + i0, 16)] = row
        return carry

    lax.fori_loop(0, _TPW // 16, step16, jnp.int32(0))
    flat_base = b * (_NUM_EXPERTS * _SPB) + w16 * _TPW
    copies = []
    for e in range(_NUM_EXPERTS):
        copies.append(pltpu.async_copy(
            oh_v.at[pl.ds(e * _TPW, _TPW)],
            out_oh.at[pl.ds(flat_base + e * _SPB, _TPW)], sem))
    copies.append(pltpu.async_copy(
        ones_v, out_ones.at[pl.ds(wid * (_T // _NW), _T // _NW)], sem))
    for c in copies:
        c.wait()


def _tc_body(out_ref):
    for b in range(_B):
        col = lax.broadcasted_iota(jnp.int32, (_NUM_EXPERTS, _TCB), 1)
        row = lax.broadcasted_iota(jnp.int32, (_NUM_EXPERTS, _TCB), 0)
        t = jnp.int32(b * _S + _SPB) + col
        c1 = lax.convert_element_type(jnp.int32(16) * t + row, jnp.uint32)
        rneg = jnp.where(row < jnp.int32(2),
                         jnp.float32(_NR_HI), jnp.float32(_NR_LO))
        key = _argmin_key(c1, row, rneg)
        idx = jnp.min(key, axis=0, keepdims=True) & jnp.int32(15)
        out_ref[b] = jnp.where(row == idx, jnp.float32(1.0), jnp.float32(0.0))


_tc_router = pl.pallas_call(
    _tc_body,
    out_shape=jax.ShapeDtypeStruct((_B, _NUM_EXPERTS, _TCB), jnp.float32),
)


def kernel(x):
    sc_oh, sc_ones = _router_sc()
    tc_oh = _tc_router()
    full = jnp.concatenate(
        [sc_oh.reshape(_B, _NUM_EXPERTS, _SPB), tc_oh], axis=2)
    one_hot = jnp.transpose(full, (0, 2, 1))
    router_probabilities = sc_ones.reshape(_B, _S, 1).astype(x.dtype)
    return (one_hot, router_probabilities, one_hot)


# speedup vs baseline: 1.1375x; 1.0141x over previous
import functools

import jax
import jax.numpy as jnp
from jax import lax
from jax.experimental import pallas as pl
from jax.experimental.pallas import tpu as pltpu
from jax.experimental.pallas import tpu_sc as plsc

_NUM_EXPERTS = 16
_B = 2
_S = 4096
_T = _B * _S
_NC = 1
_NS = 16
_NW = _NC * _NS
_WPB = _NW // _B

_SPB = 128
_TPW = _SPB // _WPB
_TCB = _S - _SPB
_UNROLL = 2

_K1 = 0x00000000
_K2 = 0x0000002A
_K3 = _K1 ^ _K2 ^ 0x1BD11BDA
_KS = (_K1, _K2, _K3)
_ROT = ((13, 15, 26, 6), (17, 29, 16, 24))

_TINY = 1.1754943508222875e-38
_LN2 = 0.6931471805599453

_NR_HI = -6.153845310211182
_NR_LO = -15.999996185302734

_LOG_COEFS = (
    0.9999951124191284,
    -0.49984779953956604,
    0.33161383867263794,
    -0.24010024964809418,
    0.16648142039775848,
    -0.09413309395313263,
    0.0354582816362381,
    -0.006320376414805651,
)


def _rotl(x, d):
    return (x << jnp.uint32(d)) | (x >> jnp.uint32(32 - d))


def _threefry_bits(c1):
    x0 = jnp.zeros_like(c1) + jnp.uint32(_KS[0])
    x1 = c1 + jnp.uint32(_KS[1])
    for i in range(5):
        for d in _ROT[i % 2]:
            x0 = x0 + x1
            x1 = _rotl(x1, d)
            x1 = x0 ^ x1
        x0 = x0 + jnp.uint32(_KS[(i + 1) % 3])
        x1 = x1 + jnp.uint32(_KS[(i + 2) % 3] + i + 1)
    return x0 ^ x1


def _vlog(v):
    bv = lax.bitcast_convert_type(v, jnp.uint32)
    e_i = lax.convert_element_type(bv >> jnp.uint32(23), jnp.int32) - jnp.int32(127)
    e = lax.convert_element_type(e_i, jnp.float32)
    m = lax.bitcast_convert_type(
        (bv & jnp.uint32(0x007FFFFF)) | jnp.uint32(0x3F800000), jnp.float32)
    x = m - jnp.float32(1.0)
    p = jnp.zeros_like(x)
    for c in _LOG_COEFS[::-1]:
        p = p * x + jnp.float32(c)
    p = p * x
    return e * jnp.float32(_LN2) + p


def _argmin_key(c1, lane, rneg):
    bits = _threefry_bits(c1)
    fb = (bits >> jnp.uint32(9)) | jnp.uint32(0x3F800000)
    f = lax.bitcast_convert_type(fb, jnp.float32) - jnp.float32(1.0)
    u = f + jnp.float32(_TINY)
    val = _vlog(u) * rneg
    return (lax.bitcast_convert_type(val, jnp.int32) & jnp.int32(-16)) | lane


def _lane_min(x, perms):
    for p in perms:
        x = jnp.minimum(x, x.at[p].get(mode="promise_in_bounds"))
    return x


@functools.partial(
    pl.kernel,
    out_type=(
        jax.ShapeDtypeStruct((_B * _NUM_EXPERTS * _SPB,), jnp.float32),
        jax.ShapeDtypeStruct((_T,), jnp.float32),
    ),
    mesh=plsc.VectorSubcoreMesh(core_axis_name="c", subcore_axis_name="s",
                                num_cores=_NC),
    scratch_types=[
        pltpu.VMEM((_NUM_EXPERTS * _TPW,), jnp.float32),
        pltpu.VMEM((_T // _NW,), jnp.float32),
        pltpu.SemaphoreType.DMA,
    ],
)
def _router_sc(out_oh, out_ones, oh_v, ones_v, sem):
    wid = lax.axis_index("s") * _NC + lax.axis_index("c")
    lane_i32 = lax.iota(jnp.int32, 16)
    lane_u32 = lax.convert_element_type(lane_i32, jnp.uint32)
    rneg = jnp.where(lane_i32 < jnp.int32(2),
                     jnp.float32(_NR_HI), jnp.float32(_NR_LO))
    perms = tuple(lane_i32 ^ jnp.int32(1 << k) for k in range(4))
    ones = jnp.full((16,), 1.0, dtype=jnp.float32)
    for j in range(_T // _NW // 16):
        ones_v[pl.ds(j * 16, 16)] = ones
    b = wid // _WPB
    w16 = wid % _WPB
    t0 = b * _S + w16 * _TPW

    def step16(s2, carry):
        i0 = s2 * 16

        def step4(s4, acc):
            jb = s4 * _UNROLL
            for k in range(_UNROLL):
                j = jb + k
                t_u32 = lax.convert_element_type(t0 + (i0 + j), jnp.uint32)
                c1 = lane_u32 + jnp.uint32(16) * t_u32
                key = _argmin_key(c1, lane_i32, rneg)
                idx = _lane_min(key, perms) & jnp.int32(15)
                acc = jnp.where(lane_i32 == j, idx, acc)
            return acc

        idxvec = lax.fori_loop(0, 16 // _UNROLL, step4,
                               jnp.zeros((16,), jnp.int32))
        for e in range(_NUM_EXPERTS):
            row = jnp.where(idxvec == jnp.int32(e),
                            jnp.float32(1.0), jnp.float32(0.0))
            oh_v[pl.ds(e * _TPW + i0, 16)] = row
        return carry

    lax.fori_loop(0, _TPW // 16, step16, jnp.int32(0))
    flat_base = b * (_NUM_EXPERTS * _SPB) + w16 * _TPW
    copies = []
    for e in range(_NUM_EXPERTS):
        copies.append(pltpu.async_copy(
            oh_v.at[pl.ds(e * _TPW, _TPW)],
            out_oh.at[pl.ds(flat_base + e * _SPB, _TPW)], sem))
    copies.append(pltpu.async_copy(
        ones_v, out_ones.at[pl.ds(wid * (_T // _NW), _T // _NW)], sem))
    for c in copies:
        c.wait()


def _tc_body(out_ref):
    for b in range(_B):
        col = lax.broadcasted_iota(jnp.int32, (_NUM_EXPERTS, _TCB), 1)
        row = lax.broadcasted_iota(jnp.int32, (_NUM_EXPERTS, _TCB), 0)
        t = jnp.int32(b * _S + _SPB) + col
        c1 = lax.convert_element_type(jnp.int32(16) * t + row, jnp.uint32)
        rneg = jnp.where(row < jnp.int32(2),
                         jnp.float32(_NR_HI), jnp.float32(_NR_LO))
        key = _argmin_key(c1, row, rneg)
        idx = jnp.min(key, axis=0, keepdims=True) & jnp.int32(15)
        out_ref[b] = jnp.where(row == idx, jnp.float32(1.0), jnp.float32(0.0))


_tc_router = pl.pallas_call(
    _tc_body,
    out_shape=jax.ShapeDtypeStruct((_B, _NUM_EXPERTS, _TCB), jnp.float32),
)


def kernel(x):
    sc_oh, sc_ones = _router_sc()
    tc_oh = _tc_router()
    full = jnp.concatenate(
        [sc_oh.reshape(_B, _NUM_EXPERTS, _SPB), tc_oh], axis=2)
    one_hot = jnp.transpose(full, (0, 2, 1))
    router_probabilities = sc_ones.reshape(_B, _S, 1).astype(x.dtype)
    return (one_hot, router_probabilities, one_hot)
